# baseline (speedup 1.0000x reference)
import functools

import jax
import jax.numpy as jnp
from jax import lax
from jax.experimental import pallas as pl
from jax.experimental.pallas import tpu as pltpu
from jax.experimental.pallas import tpu_sc as plsc

B_TOTAL = 1024 * 200
D = 64
NC, NS, L = 2, 16, 16
NW = NC * NS
B_PER_W = B_TOTAL // NW
CHUNK = 128
NCHUNK = B_PER_W // CHUNK
SCALE = 8.0

_mesh = plsc.VectorSubcoreMesh(
    core_axis_name="c", subcore_axis_name="s", num_cores=NC, num_subcores=NS
)


@functools.partial(
    pl.kernel,
    out_type=jax.ShapeDtypeStruct((B_TOTAL, D), jnp.float32),
    mesh=_mesh,
    scratch_types=[
        pltpu.VMEM((NCHUNK, CHUNK), jnp.int32),
        pltpu.VMEM((CHUNK, D), jnp.float32),
        pltpu.VMEM((CHUNK, D), jnp.float32),
        pltpu.SemaphoreType.DMA,
    ],
    compiler_params=pltpu.CompilerParams(use_tc_tiling_on_sc=False),
)
def _embed(x_hbm, table_hbm, out_hbm, idx_v, rows_in, rows_out, sem):
    wid = lax.axis_index("s") * NC + lax.axis_index("c")
    base = wid * B_PER_W
    pltpu.sync_copy(x_hbm.at[wid], idx_v)

    @pl.loop(0, NCHUNK)
    def _chunk(c):
        pltpu.async_copy(table_hbm.at[idx_v.at[c]], rows_in, sem).wait()

        @pl.loop(0, CHUNK)
        def _scale(i):
            for j in range(D // L):
                sl = pl.ds(j * L, L)
                rows_out[i, sl] = rows_in[i, sl] * SCALE

        pltpu.sync_copy(rows_out, out_hbm.at[pl.ds(base + c * CHUNK, CHUNK)])


def kernel(x, table):
    xf = x.reshape(NW, NCHUNK, CHUNK)
    out = _embed(xf, table)
    return out.reshape(x.shape[0], x.shape[1], D)
